# baseline (speedup 1.0000x reference)
import functools

import jax
import jax.numpy as jnp
from jax import lax
from jax.experimental import pallas as pl
from jax.experimental.pallas import tpu as pltpu
from jax.experimental.pallas import tpu_sc as plsc

N_ATOMS = 10000
N_BONDS = 320000
MAX_NB = 32
ATOM_FDIM = 128
BOND_FDIM = 144
HIDDEN = 128
N_MOLS = 200

_NC = 2
_NS = 16
_NW = _NC * _NS

_APW = 384
A_PAD = _NW * _APW
_BPW = N_BONDS // _NW
_CB = 80
_NCH = _BPW // _CB

_MESH = dict(
    mesh=plsc.VectorSubcoreMesh(core_axis_name="c", subcore_axis_name="s"),
)


def _wid():
    return lax.axis_index("s") * _NC + lax.axis_index("c")


_AC = 4
_ACH = _APW // _AC
_ROWS = _AC * MAX_NB


def _s1_body(negate, relu_in, tab, a2bf, out, idx_v,
             stg0, stg1, stg2, ob0, ob1, ob2,
             sg0, sg1, sg2, so0, so1, so2):
    base = _wid() * _APW
    pltpu.sync_copy(a2bf.at[pl.ds(base * MAX_NB, _APW * MAX_NB)], idx_v)
    stg = (stg0, stg1, stg2)
    obuf = (ob0, ob1, ob2)
    sg = (sg0, sg1, sg2)
    so = (so0, so1, so2)

    def fire(c, u):
        pltpu.async_copy(tab.at[idx_v.at[pl.ds(c * _ROWS, _ROWS)]],
                         stg[u], sg[u])

    def drain(c, u):
        pltpu.make_async_copy(tab.at[idx_v.at[pl.ds(c * _ROWS, _ROWS)]],
                              stg[u], sg[u]).wait()

    def alu(u):
        def atom(a, _):
            r0 = a * MAX_NB
            for g in range(HIDDEN // 16):
                sl = pl.ds(g * 16, 16)
                vals = [stg[u][r0 + j, sl] for j in range(MAX_NB)]
                if relu_in:
                    vals = [jnp.maximum(v, 0.0) for v in vals]
                while len(vals) > 1:
                    vals = [vals[i] + vals[i + 1]
                            for i in range(0, len(vals), 2)]
                obuf[u][a, sl] = -vals[0] if negate else vals[0]
            return 0
        lax.fori_loop(0, _AC, atom, 0)

    def waitO(u):
        pltpu.make_async_copy(obuf[u], out.at[pl.ds(0, _AC)], so[u]).wait()

    fire(0, 0)
    fire(1, 1)

    def triple(tt, _):
        c0 = 3 * tt
        for u in range(3):
            c = c0 + u
            up = (u + 2) % 3
            if u == 0:
                fire(c + 2, up)
            else:
                @pl.when(tt < _ACH // 3 - 1)
                def _():
                    fire(c + 2, up)
            drain(c, u)
            @pl.when(tt >= 1)
            def _():
                waitO(u)
            alu(u)
            pltpu.async_copy(obuf[u], out.at[pl.ds(base + c * _AC, _AC)],
                             so[u])
        return 0
    lax.fori_loop(0, _ACH // 3, triple, 0)
    waitO(0)
    waitO(1)
    waitO(2)


def _make_s1(negate, relu_in):
    return pl.kernel(
        functools.partial(_s1_body, negate, relu_in),
        out_type=jax.ShapeDtypeStruct((A_PAD, HIDDEN), jnp.float32),
        scratch_types=[
            pltpu.VMEM((_APW * MAX_NB,), jnp.int32),
            pltpu.VMEM((_ROWS, HIDDEN), jnp.float32),
            pltpu.VMEM((_ROWS, HIDDEN), jnp.float32),
            pltpu.VMEM((_ROWS, HIDDEN), jnp.float32),
            pltpu.VMEM((_AC, HIDDEN), jnp.float32),
            pltpu.VMEM((_AC, HIDDEN), jnp.float32),
            pltpu.VMEM((_AC, HIDDEN), jnp.float32),
            pltpu.SemaphoreType.DMA,
            pltpu.SemaphoreType.DMA,
            pltpu.SemaphoreType.DMA,
            pltpu.SemaphoreType.DMA,
            pltpu.SemaphoreType.DMA,
            pltpu.SemaphoreType.DMA,
        ],
        **_MESH,
    )


_s1_neg = _make_s1(True, False)
_s1_pos = _make_s1(False, True)


def _s2_body(inp, amh, mhn, b2a, brev, out,
             ia_v, ir_v, b0, b1, b2,
             sl0, sl1, sl2, sa0, sa1, sa2, so0, so1, so2):
    bbase = _wid() * _BPW
    pltpu.sync_copy(b2a.at[pl.ds(bbase, _BPW)], ia_v)
    pltpu.sync_copy(brev.at[pl.ds(bbase, _BPW)], ir_v)
    bufs = (b0, b1, b2)
    sl = (sl0, sl1, sl2)
    sa = (sa0, sa1, sa2)
    so = (so0, so1, so2)

    def L(c, u):
        pltpu.async_copy(inp.at[pl.ds(bbase + c * _CB, _CB)], bufs[u], sl[u])

    def waitL(c, u):
        pltpu.make_async_copy(inp.at[pl.ds(bbase + c * _CB, _CB)],
                              bufs[u], sl[u]).wait()

    def A(c, u):
        off = c * _CB
        pltpu.async_copy(amh.at[ia_v.at[pl.ds(off, _CB)]], bufs[u], sa[u],
                         add=True)
        pltpu.async_copy(mhn.at[ir_v.at[pl.ds(off, _CB)]], bufs[u], sa[u],
                         add=True)

    def waitA(c, u):
        off = c * _CB
        pltpu.make_async_copy(amh.at[ia_v.at[pl.ds(off, _CB)]], bufs[u],
                              sa[u]).wait()
        pltpu.make_async_copy(mhn.at[ir_v.at[pl.ds(off, _CB)]], bufs[u],
                              sa[u]).wait()

    def O(c, u):
        pltpu.async_copy(bufs[u], out.at[pl.ds(bbase + c * _CB, _CB)], so[u])

    def waitO(u):
        pltpu.make_async_copy(bufs[u], out.at[pl.ds(0, _CB)], so[u]).wait()

    def step(c, u, has_next, has_next2, has_prev):
        un = (u + 1) % 3
        up = (u + 2) % 3
        if has_next:
            waitL(c + 1, un)
            A(c + 1, un)
        if has_prev is True:
            waitO(up)
        elif has_prev is not False:
            @pl.when(has_prev)
            def _():
                waitO(up)
        if has_next2:
            L(c + 2, up)
        waitA(c, u)
        O(c, u)

    L(0, 0)
    L(1, 1)
    waitL(0, 0)
    A(0, 0)

    def triple(tt, _):
        c0 = 3 * tt
        for u in range(3):
            c = c0 + u
            step(c, u, True, True, True if u > 0 else (tt > 0))
        return 0
    lax.fori_loop(0, (_NCH - 2) // 3, triple, 0)
    step(_NCH - 2, (_NCH - 2) % 3, True, False, True)
    step(_NCH - 1, (_NCH - 1) % 3, False, False, True)
    waitO((_NCH - 1) % 3)


_s2 = pl.kernel(
    _s2_body,
    out_type=jax.ShapeDtypeStruct((N_BONDS, HIDDEN), jnp.float32),
    scratch_types=[
        pltpu.VMEM((_BPW,), jnp.int32),
        pltpu.VMEM((_BPW,), jnp.int32),
        pltpu.VMEM((_CB, HIDDEN), jnp.float32),
        pltpu.VMEM((_CB, HIDDEN), jnp.float32),
        pltpu.VMEM((_CB, HIDDEN), jnp.float32),
        pltpu.SemaphoreType.DMA,
        pltpu.SemaphoreType.DMA,
        pltpu.SemaphoreType.DMA,
        pltpu.SemaphoreType.DMA,
        pltpu.SemaphoreType.DMA,
        pltpu.SemaphoreType.DMA,
        pltpu.SemaphoreType.DMA,
        pltpu.SemaphoreType.DMA,
        pltpu.SemaphoreType.DMA,
    ],
    **_MESH,
)


_BB = 3200


def _k1_body(fb_ref, wi_ref, whn_ref, inp_ref, mhn_ref):
    x = jnp.dot(fb_ref[...].astype(jnp.bfloat16), wi_ref[...],
                preferred_element_type=jnp.float32)
    inp_ref[...] = x
    mhn_ref[...] = jnp.dot(jnp.maximum(x, 0.0).astype(jnp.bfloat16),
                           whn_ref[...],
                           preferred_element_type=jnp.float32)


_k1 = pl.pallas_call(
    _k1_body,
    grid=(N_BONDS // _BB,),
    in_specs=[
        pl.BlockSpec((_BB, BOND_FDIM), lambda i: (i, 0)),
        pl.BlockSpec((BOND_FDIM, HIDDEN), lambda i: (0, 0)),
        pl.BlockSpec((HIDDEN, HIDDEN), lambda i: (0, 0)),
    ],
    out_specs=[
        pl.BlockSpec((_BB, HIDDEN), lambda i: (i, 0)),
        pl.BlockSpec((_BB, HIDDEN), lambda i: (i, 0)),
    ],
    out_shape=[jax.ShapeDtypeStruct((N_BONDS, HIDDEN), jnp.float32)] * 2,
    compiler_params=pltpu.CompilerParams(
        dimension_semantics=("arbitrary",)),
)


def _k2_body(msg_ref, whn_ref, mhn_ref):
    mhn_ref[...] = jnp.dot(jnp.maximum(msg_ref[...], 0.0), whn_ref[...],
                           preferred_element_type=jnp.float32)


_k2 = pl.pallas_call(
    _k2_body,
    grid=(N_BONDS // _BB,),
    in_specs=[
        pl.BlockSpec((_BB, HIDDEN), lambda i: (i, 0)),
        pl.BlockSpec((HIDDEN, HIDDEN), lambda i: (0, 0)),
    ],
    out_specs=pl.BlockSpec((_BB, HIDDEN), lambda i: (i, 0)),
    out_shape=jax.ShapeDtypeStruct((N_BONDS, HIDDEN), jnp.float32),
    compiler_params=pltpu.CompilerParams(
        dimension_semantics=("arbitrary",)),
)


_AB = 2000
_AGRID = N_ATOMS // _AB


def _k3_body(fa_ref, am_ref, mi_ref, wo1_ref, wo2_ref, bo_ref, out_ref,
             sums, counts):
    i = pl.program_id(0)

    @pl.when(i == 0)
    def _():
        sums[...] = jnp.zeros_like(sums)
        counts[...] = jnp.zeros_like(counts)

    hid = jnp.maximum(
        jnp.dot(fa_ref[...], wo1_ref[...], preferred_element_type=jnp.float32)
        + jnp.dot(am_ref[...], wo2_ref[...], preferred_element_type=jnp.float32)
        + bo_ref[...], 0.0)
    mi = mi_ref[0]
    seg = lax.broadcasted_iota(jnp.int32, (N_MOLS, _AB), 0)
    mask = (seg == mi).astype(jnp.float32)
    sums[...] += jnp.dot(mask, hid, preferred_element_type=jnp.float32)
    counts[...] += jnp.broadcast_to(
        jnp.sum(mask, axis=1, keepdims=True), (N_MOLS, HIDDEN))

    @pl.when(i == pl.num_programs(0) - 1)
    def _():
        out_ref[...] = sums[...] / jnp.maximum(counts[...], 1.0)


_k3 = pl.pallas_call(
    _k3_body,
    grid=(_AGRID,),
    in_specs=[
        pl.BlockSpec((_AB, ATOM_FDIM), lambda i: (i, 0)),
        pl.BlockSpec((_AB, HIDDEN), lambda i: (i, 0)),
        pl.BlockSpec((1, 1, _AB), lambda i: (i, 0, 0)),
        pl.BlockSpec((ATOM_FDIM, HIDDEN), lambda i: (0, 0)),
        pl.BlockSpec((HIDDEN, HIDDEN), lambda i: (0, 0)),
        pl.BlockSpec((1, HIDDEN), lambda i: (0, 0)),
    ],
    out_specs=pl.BlockSpec((N_MOLS, HIDDEN), lambda i: (0, 0)),
    out_shape=jax.ShapeDtypeStruct((N_MOLS, HIDDEN), jnp.float32),
    scratch_shapes=[
        pltpu.VMEM((N_MOLS, HIDDEN), jnp.float32),
        pltpu.VMEM((N_MOLS, HIDDEN), jnp.float32),
    ],
    compiler_params=pltpu.CompilerParams(
        dimension_semantics=("arbitrary",)),
)


def kernel(f_atoms, f_bonds, a2b, b2a, b2revb, mol_ids, W_i, W_h, W_o, b_o):
    Wi16 = W_i.astype(jnp.bfloat16)
    Whn16 = (-W_h).astype(jnp.bfloat16)
    W_hn = -W_h
    npad = (A_PAD - N_ATOMS) * MAX_NB
    pad_idx = (jnp.arange(npad, dtype=jnp.int32) * 97) % N_BONDS
    a2bf = jnp.concatenate([a2b.reshape(-1), pad_idx], axis=0)
    mol3 = mol_ids.reshape(_AGRID, 1, _AB)
    bo2 = b_o.reshape(1, HIDDEN)

    inp, mhn = _k1(f_bonds, Wi16, Whn16)
    amh = _s1_neg(mhn, a2bf)
    msg = _s2(inp, amh, mhn, b2a, b2revb)
    mhn = _k2(msg, W_hn)
    amh = _s1_neg(mhn, a2bf)
    msg = _s2(inp, amh, mhn, b2a, b2revb)
    amsg = _s1_pos(msg, a2bf)
    return _k3(f_atoms, amsg, mol3, W_o[:ATOM_FDIM], W_o[ATOM_FDIM:], bo2)


# speedup vs baseline: 1.0127x; 1.0127x over previous
import functools

import jax
import jax.numpy as jnp
from jax import lax
from jax.experimental import pallas as pl
from jax.experimental.pallas import tpu as pltpu
from jax.experimental.pallas import tpu_sc as plsc

N_ATOMS = 10000
N_BONDS = 320000
MAX_NB = 32
ATOM_FDIM = 128
BOND_FDIM = 144
HIDDEN = 128
N_MOLS = 200

_NC = 2
_NS = 16
_NW = _NC * _NS

_APW = 384
A_PAD = _NW * _APW
_BPW = N_BONDS // _NW
_CB = 80
_NCH = _BPW // _CB

_MESH = dict(
    mesh=plsc.VectorSubcoreMesh(core_axis_name="c", subcore_axis_name="s"),
)


def _wid():
    return lax.axis_index("s") * _NC + lax.axis_index("c")


_AC = 4
_ACH = _APW // _AC
_ROWS = _AC * MAX_NB


def _s1_body(negate, relu_in, tab, a2bf, out, idx_v,
             stg0, stg1, stg2, ob0, ob1, ob2,
             sg0, sg1, sg2, so0, so1, so2):
    base = _wid() * _APW
    pltpu.sync_copy(a2bf.at[pl.ds(base * MAX_NB, _APW * MAX_NB)], idx_v)
    stg = (stg0, stg1, stg2)
    obuf = (ob0, ob1, ob2)
    sg = (sg0, sg1, sg2)
    so = (so0, so1, so2)

    def fire(c, u):
        pltpu.async_copy(tab.at[idx_v.at[pl.ds(c * _ROWS, _ROWS)]],
                         stg[u], sg[u])

    def drain(c, u):
        pltpu.make_async_copy(tab.at[idx_v.at[pl.ds(c * _ROWS, _ROWS)]],
                              stg[u], sg[u]).wait()

    def alu(u):
        def atom(a, _):
            r0 = a * MAX_NB
            for g in range(HIDDEN // 16):
                sl = pl.ds(g * 16, 16)
                vals = [stg[u][r0 + j, sl] for j in range(MAX_NB)]
                if relu_in:
                    vals = [jnp.maximum(v, 0.0) for v in vals]
                while len(vals) > 1:
                    vals = [vals[i] + vals[i + 1]
                            for i in range(0, len(vals), 2)]
                obuf[u][a, sl] = -vals[0] if negate else vals[0]
            return 0
        lax.fori_loop(0, _AC, atom, 0)

    def waitO(u):
        pltpu.make_async_copy(obuf[u], out.at[pl.ds(0, _AC)], so[u]).wait()

    fire(0, 0)
    fire(1, 1)

    def triple(tt, _):
        c0 = 3 * tt
        for u in range(3):
            c = c0 + u
            up = (u + 2) % 3
            if u == 0:
                fire(c + 2, up)
            else:
                @pl.when(tt < _ACH // 3 - 1)
                def _():
                    fire(c + 2, up)
            drain(c, u)
            @pl.when(tt >= 1)
            def _():
                waitO(u)
            alu(u)
            pltpu.async_copy(obuf[u], out.at[pl.ds(base + c * _AC, _AC)],
                             so[u])
        return 0
    lax.fori_loop(0, _ACH // 3, triple, 0)
    waitO(0)
    waitO(1)
    waitO(2)


def _make_s1(negate, relu_in):
    return pl.kernel(
        functools.partial(_s1_body, negate, relu_in),
        out_type=jax.ShapeDtypeStruct((A_PAD, HIDDEN), jnp.float32),
        scratch_types=[
            pltpu.VMEM((_APW * MAX_NB,), jnp.int32),
            pltpu.VMEM((_ROWS, HIDDEN), jnp.float32),
            pltpu.VMEM((_ROWS, HIDDEN), jnp.float32),
            pltpu.VMEM((_ROWS, HIDDEN), jnp.float32),
            pltpu.VMEM((_AC, HIDDEN), jnp.float32),
            pltpu.VMEM((_AC, HIDDEN), jnp.float32),
            pltpu.VMEM((_AC, HIDDEN), jnp.float32),
            pltpu.SemaphoreType.DMA,
            pltpu.SemaphoreType.DMA,
            pltpu.SemaphoreType.DMA,
            pltpu.SemaphoreType.DMA,
            pltpu.SemaphoreType.DMA,
            pltpu.SemaphoreType.DMA,
        ],
        **_MESH,
    )


_s1_neg = _make_s1(True, False)
_s1_pos = _make_s1(False, True)


def _s2_body(inp, amh, mhn, b2a, brev, out,
             ia_v, ir_v, b0, b1, b2,
             sl0, sl1, sl2, sa0, sa1, sa2, so0, so1, so2):
    bbase = _wid() * _BPW
    pltpu.sync_copy(b2a.at[pl.ds(bbase, _BPW)], ia_v)
    pltpu.sync_copy(brev.at[pl.ds(bbase, _BPW)], ir_v)
    bufs = (b0, b1, b2)
    sl = (sl0, sl1, sl2)
    sa = (sa0, sa1, sa2)
    so = (so0, so1, so2)

    def L(c, u):
        pltpu.async_copy(inp.at[pl.ds(bbase + c * _CB, _CB)], bufs[u], sl[u])

    def waitL(c, u):
        pltpu.make_async_copy(inp.at[pl.ds(bbase + c * _CB, _CB)],
                              bufs[u], sl[u]).wait()

    def A(c, u):
        off = c * _CB
        pltpu.async_copy(amh.at[ia_v.at[pl.ds(off, _CB)]], bufs[u], sa[u],
                         add=True)
        pltpu.async_copy(mhn.at[ir_v.at[pl.ds(off, _CB)]], bufs[u], sa[u],
                         add=True)

    def waitA(c, u):
        off = c * _CB
        pltpu.make_async_copy(amh.at[ia_v.at[pl.ds(off, _CB)]], bufs[u],
                              sa[u]).wait()
        pltpu.make_async_copy(mhn.at[ir_v.at[pl.ds(off, _CB)]], bufs[u],
                              sa[u]).wait()

    def O(c, u):
        pltpu.async_copy(bufs[u], out.at[pl.ds(bbase + c * _CB, _CB)], so[u])

    def waitO(u):
        pltpu.make_async_copy(bufs[u], out.at[pl.ds(0, _CB)], so[u]).wait()

    def step(c, u, has_next, has_next2, has_prev):
        un = (u + 1) % 3
        up = (u + 2) % 3
        if has_next:
            waitL(c + 1, un)
            A(c + 1, un)
        if has_prev is True:
            waitO(up)
        elif has_prev is not False:
            @pl.when(has_prev)
            def _():
                waitO(up)
        if has_next2:
            L(c + 2, up)
        waitA(c, u)
        O(c, u)

    L(0, 0)
    L(1, 1)
    waitL(0, 0)
    A(0, 0)

    def triple(tt, _):
        c0 = 3 * tt
        for u in range(3):
            c = c0 + u
            step(c, u, True, True, True if u > 0 else (tt > 0))
        return 0
    lax.fori_loop(0, (_NCH - 2) // 3, triple, 0)
    step(_NCH - 2, (_NCH - 2) % 3, True, False, True)
    step(_NCH - 1, (_NCH - 1) % 3, False, False, True)
    waitO((_NCH - 1) % 3)


_s2 = pl.kernel(
    _s2_body,
    out_type=jax.ShapeDtypeStruct((N_BONDS, HIDDEN), jnp.float32),
    scratch_types=[
        pltpu.VMEM((_BPW,), jnp.int32),
        pltpu.VMEM((_BPW,), jnp.int32),
        pltpu.VMEM((_CB, HIDDEN), jnp.float32),
        pltpu.VMEM((_CB, HIDDEN), jnp.float32),
        pltpu.VMEM((_CB, HIDDEN), jnp.float32),
        pltpu.SemaphoreType.DMA,
        pltpu.SemaphoreType.DMA,
        pltpu.SemaphoreType.DMA,
        pltpu.SemaphoreType.DMA,
        pltpu.SemaphoreType.DMA,
        pltpu.SemaphoreType.DMA,
        pltpu.SemaphoreType.DMA,
        pltpu.SemaphoreType.DMA,
        pltpu.SemaphoreType.DMA,
    ],
    **_MESH,
)


_BB = 3200


def _k1_body(fb_ref, wi_ref, whn_ref, inp_ref, mhn_ref):
    x = jnp.dot(fb_ref[...], wi_ref[...], preferred_element_type=jnp.float32)
    inp_ref[...] = x
    mhn_ref[...] = jnp.dot(jnp.maximum(x, 0.0).astype(jnp.bfloat16),
                           whn_ref[...],
                           preferred_element_type=jnp.float32)


_k1 = pl.pallas_call(
    _k1_body,
    grid=(N_BONDS // _BB,),
    in_specs=[
        pl.BlockSpec((_BB, BOND_FDIM), lambda i: (i, 0)),
        pl.BlockSpec((BOND_FDIM, HIDDEN), lambda i: (0, 0)),
        pl.BlockSpec((HIDDEN, HIDDEN), lambda i: (0, 0)),
    ],
    out_specs=[
        pl.BlockSpec((_BB, HIDDEN), lambda i: (i, 0)),
        pl.BlockSpec((_BB, HIDDEN), lambda i: (i, 0)),
    ],
    out_shape=[jax.ShapeDtypeStruct((N_BONDS, HIDDEN), jnp.float32)] * 2,
    compiler_params=pltpu.CompilerParams(
        dimension_semantics=("arbitrary",)),
)


def _k2_body(msg_ref, whn_ref, mhn_ref):
    mhn_ref[...] = jnp.dot(jnp.maximum(msg_ref[...], 0.0), whn_ref[...],
                           preferred_element_type=jnp.float32)


_k2 = pl.pallas_call(
    _k2_body,
    grid=(N_BONDS // _BB,),
    in_specs=[
        pl.BlockSpec((_BB, HIDDEN), lambda i: (i, 0)),
        pl.BlockSpec((HIDDEN, HIDDEN), lambda i: (0, 0)),
    ],
    out_specs=pl.BlockSpec((_BB, HIDDEN), lambda i: (i, 0)),
    out_shape=jax.ShapeDtypeStruct((N_BONDS, HIDDEN), jnp.float32),
    compiler_params=pltpu.CompilerParams(
        dimension_semantics=("arbitrary",)),
)


_AB = 2000
_AGRID = N_ATOMS // _AB


def _k3_body(fa_ref, am_ref, mi_ref, wo1_ref, wo2_ref, bo_ref, out_ref,
             sums, counts):
    i = pl.program_id(0)

    @pl.when(i == 0)
    def _():
        sums[...] = jnp.zeros_like(sums)
        counts[...] = jnp.zeros_like(counts)

    hid = jnp.maximum(
        jnp.dot(fa_ref[...], wo1_ref[...], preferred_element_type=jnp.float32)
        + jnp.dot(am_ref[...], wo2_ref[...], preferred_element_type=jnp.float32)
        + bo_ref[...], 0.0)
    mi = mi_ref[0]
    seg = lax.broadcasted_iota(jnp.int32, (N_MOLS, _AB), 0)
    mask = (seg == mi).astype(jnp.float32)
    sums[...] += jnp.dot(mask, hid, preferred_element_type=jnp.float32)
    counts[...] += jnp.broadcast_to(
        jnp.sum(mask, axis=1, keepdims=True), (N_MOLS, HIDDEN))

    @pl.when(i == pl.num_programs(0) - 1)
    def _():
        out_ref[...] = sums[...] / jnp.maximum(counts[...], 1.0)


_k3 = pl.pallas_call(
    _k3_body,
    grid=(_AGRID,),
    in_specs=[
        pl.BlockSpec((_AB, ATOM_FDIM), lambda i: (i, 0)),
        pl.BlockSpec((_AB, HIDDEN), lambda i: (i, 0)),
        pl.BlockSpec((1, 1, _AB), lambda i: (i, 0, 0)),
        pl.BlockSpec((ATOM_FDIM, HIDDEN), lambda i: (0, 0)),
        pl.BlockSpec((HIDDEN, HIDDEN), lambda i: (0, 0)),
        pl.BlockSpec((1, HIDDEN), lambda i: (0, 0)),
    ],
    out_specs=pl.BlockSpec((N_MOLS, HIDDEN), lambda i: (0, 0)),
    out_shape=jax.ShapeDtypeStruct((N_MOLS, HIDDEN), jnp.float32),
    scratch_shapes=[
        pltpu.VMEM((N_MOLS, HIDDEN), jnp.float32),
        pltpu.VMEM((N_MOLS, HIDDEN), jnp.float32),
    ],
    compiler_params=pltpu.CompilerParams(
        dimension_semantics=("arbitrary",)),
)


def kernel(f_atoms, f_bonds, a2b, b2a, b2revb, mol_ids, W_i, W_h, W_o, b_o):
    fb16 = f_bonds.astype(jnp.bfloat16)
    Wi16 = W_i.astype(jnp.bfloat16)
    Whn16 = (-W_h).astype(jnp.bfloat16)
    W_hn = -W_h
    npad = (A_PAD - N_ATOMS) * MAX_NB
    pad_idx = (jnp.arange(npad, dtype=jnp.int32) * 97) % N_BONDS
    a2bf = jnp.concatenate([a2b.reshape(-1), pad_idx], axis=0)
    mol3 = mol_ids.reshape(_AGRID, 1, _AB)
    bo2 = b_o.reshape(1, HIDDEN)

    inp, mhn = _k1(fb16, Wi16, Whn16)
    amh = _s1_neg(mhn, a2bf)
    msg = _s2(inp, amh, mhn, b2a, b2revb)
    mhn = _k2(msg, W_hn)
    amh = _s1_neg(mhn, a2bf)
    msg = _s2(inp, amh, mhn, b2a, b2revb)
    amsg = _s1_pos(msg, a2bf)
    return _k3(f_atoms, amsg, mol3, W_o[:ATOM_FDIM], W_o[ATOM_FDIM:], bo2)
